# Initial kernel scaffold; baseline (speedup 1.0000x reference)
import jax
import jax.numpy as jnp
from jax.experimental import pallas as pl
from jax.experimental.pallas import tpu as pltpu

_THR = 0.18
_B, _C, _H, _W = 2, 96, 512, 512
_HR = _H // 2
_BR = 32
_NR = _HR // _BR


def _body(sig_ref, mask_ref, val_ref, p_ref):
    r = pl.program_id(1)
    s = sig_ref[0]
    hm = jnp.mean(s, axis=0)

    rr = r * _BR + jax.lax.broadcasted_iota(jnp.int32, (_BR, _W), 0)
    col = jax.lax.broadcasted_iota(jnp.int32, (_BR, _W), 1)
    patt = jnp.where((rr % 2) == 0, (col % 4) == 1, (col % 4) == 3)
    p = jnp.where((hm <= _THR) & patt, 1.0, 0.0)
    p_ref[pl.ds(r * _BR, _BR), :] = p

    @pl.when(r == _NR - 1)
    def _tail():
        pf = p_ref[...]
        colh = jax.lax.broadcasted_iota(jnp.int32, (_HR, _W), 1)
        rowh = jax.lax.broadcasted_iota(jnp.int32, (_HR, _W), 0)
        even_col = (colh % 2) == 0
        psr = jnp.roll(pf, 1, axis=1) * jnp.where(colh > 0, 1.0, 0.0)
        psl = jnp.roll(pf, -1, axis=1)
        pdn = jnp.roll(pf, -1, axis=0) * jnp.where(rowh < _HR - 1, 1.0, 0.0)

        me = jnp.where(even_col, 1.0 - psr, 0.0)
        mo = jnp.where(even_col, 0.0, 1.0 - jnp.maximum(pf, pdn))
        mask_ref[0, 0, :, 0:_W] = me
        mask_ref[0, 0, :, _W:] = mo
        val_ref[0, 0, :, 0:_W] = psl
        val_ref[0, 0, :, _W:] = jnp.zeros((_HR, _W), jnp.float32)


@jax.jit
def kernel(sigma):
    sig4 = sigma.reshape(_B, _C, _HR, 2 * _W)
    out_sds = jax.ShapeDtypeStruct((_B, 1, _HR, 2 * _W), jnp.float32)
    mask, values = pl.pallas_call(
        _body,
        grid=(_B, _NR),
        in_specs=[pl.BlockSpec((1, _C, _BR, _W), lambda b, r: (b, 0, r, 0))],
        out_specs=[
            pl.BlockSpec((1, 1, _HR, 2 * _W), lambda b, r: (b, 0, 0, 0)),
            pl.BlockSpec((1, 1, _HR, 2 * _W), lambda b, r: (b, 0, 0, 0)),
        ],
        out_shape=[out_sds, out_sds],
        scratch_shapes=[pltpu.VMEM((_HR, _W), jnp.float32)],
        compiler_params=pltpu.CompilerParams(
            dimension_semantics=("arbitrary", "arbitrary"),
        ),
    )(sig4)
    return mask.reshape(_B, 1, _H, _W), values.reshape(_B, 1, _H, _W)


# speedup vs baseline: 6.8452x; 6.8452x over previous
import jax
import jax.numpy as jnp
from jax.experimental import pallas as pl
from jax.experimental.pallas import tpu as pltpu

_THR = 0.18
_B, _C, _H, _W = 2, 96, 512, 512
_HR = _H // 2
_BR = 32
_NR = _HR // _BR


def _body(sig_ref, mask_ref, val_ref, p_ref):
    r = pl.program_id(1)
    s = sig_ref[0]
    hm = jnp.mean(s, axis=0)

    rr = r * _BR + jax.lax.broadcasted_iota(jnp.int32, (_BR, _W), 0)
    col = jax.lax.broadcasted_iota(jnp.int32, (_BR, _W), 1)
    pat1 = jnp.where((col % 4) == 1, 1.0, 0.0)
    pat3 = jnp.where((col % 4) == 3, 1.0, 0.0)
    row_even = jnp.where((rr % 2) == 0, 1.0, 0.0)
    patt = row_even * pat1 + (1.0 - row_even) * pat3
    below = jnp.where(hm <= _THR, 1.0, 0.0)
    p_ref[pl.ds(r * _BR, _BR), :] = below * patt

    @pl.when(r == _NR - 1)
    def _tail():
        pf = p_ref[...]
        colh = jax.lax.broadcasted_iota(jnp.int32, (_HR, _W), 1)
        rowh = jax.lax.broadcasted_iota(jnp.int32, (_HR, _W), 0)
        even_col = jnp.where((colh % 2) == 0, 1.0, 0.0)
        psr = jnp.roll(pf, 1, axis=1) * jnp.where(colh > 0, 1.0, 0.0)
        psl = jnp.roll(pf, -1, axis=1)
        pdn = jnp.roll(pf, -1, axis=0) * jnp.where(rowh < _HR - 1, 1.0, 0.0)

        me = even_col * (1.0 - psr)
        mo = (1.0 - even_col) * (1.0 - jnp.maximum(pf, pdn))
        mask_ref[0, 0, :, 0:_W] = me
        mask_ref[0, 0, :, _W:] = mo
        val_ref[0, 0, :, 0:_W] = psl
        val_ref[0, 0, :, _W:] = jnp.zeros((_HR, _W), jnp.float32)


@jax.jit
def kernel(sigma):
    sig4 = sigma.reshape(_B, _C, _HR, 2 * _W)
    out_sds = jax.ShapeDtypeStruct((_B, 1, _HR, 2 * _W), jnp.float32)
    mask, values = pl.pallas_call(
        _body,
        grid=(_B, _NR),
        in_specs=[pl.BlockSpec((1, _C, _BR, _W), lambda b, r: (b, 0, r, 0))],
        out_specs=[
            pl.BlockSpec((1, 1, _HR, 2 * _W), lambda b, r: (b, 0, 0, 0)),
            pl.BlockSpec((1, 1, _HR, 2 * _W), lambda b, r: (b, 0, 0, 0)),
        ],
        out_shape=[out_sds, out_sds],
        scratch_shapes=[pltpu.VMEM((_HR, _W), jnp.float32)],
        compiler_params=pltpu.CompilerParams(
            dimension_semantics=("arbitrary", "arbitrary"),
        ),
    )(sig4)
    return mask.reshape(_B, 1, _H, _W), values.reshape(_B, 1, _H, _W)
